# Initial kernel scaffold; baseline (speedup 1.0000x reference)
import jax
import jax.numpy as jnp
from jax.experimental import pallas as pl


def kernel(x, edge_index, edge_attr, batch, W1, b1, W2, b2, gn1_w, gn1_b, gn1_a, gn2_w, gn2_b, gn2_a, p1, p0, p2, lin1_w, lin1_b, lin2_w, lin2_b, lin3_w, lin3_b):
    raise NotImplementedError("write your pallas kernel here")


# speedup vs baseline: 1.2520x; 1.2520x over previous
import functools

import jax
import jax.numpy as jnp
from jax import lax
from jax.experimental import pallas as pl
from jax.experimental.pallas import tpu as pltpu

N = 50000
E = 800000
G = 128
EPS = 1e-5


def _prelu(x, w):
    return jnp.where(x >= 0, x, w * x)


def _head_body(pooled_ref, l1w_ref, l1b_ref, l2w_ref, l2b_ref, l3w_ref,
               l3b_ref, p0_ref, p2_ref, out_ref):
    h = pooled_ref[...]
    h = h @ l1w_ref[...] + l1b_ref[...][None, :]
    h = jnp.where(h >= 0, h, p0_ref[...][None, :] * h)
    h = h @ l2w_ref[...] + l2b_ref[...][None, :]
    h = jnp.where(h >= 0, h, p2_ref[...][None, :] * h)
    h = h @ l3w_ref[...] + l3b_ref[...][None, :]
    out_ref[...] = h


def _head(pooled, lin1_w, lin1_b, lin2_w, lin2_b, lin3_w, lin3_b, p0, p2):
    return pl.pallas_call(
        _head_body,
        out_shape=jax.ShapeDtypeStruct((G, 1), jnp.float32),
    )(pooled, lin1_w, lin1_b, lin2_w, lin2_b, lin3_w, lin3_b, p0, p2)


def kernel(x, edge_index, edge_attr, batch, W1, b1, W2, b2, gn1_w, gn1_b,
           gn1_a, gn2_w, gn2_b, gn2_a, p1, p0, p2, lin1_w, lin1_b, lin2_w,
           lin2_b, lin3_w, lin3_b):
    from jax.ops import segment_sum, segment_max

    row = edge_index[0]
    col = edge_index[1]
    deg = segment_sum(edge_attr, col, num_segments=N) + 1.0
    dinv = lax.rsqrt(deg)
    norm = dinv[row] * edge_attr * dinv[col]

    def conv(h, W, b):
        xw = h @ W
        out = segment_sum(norm[:, None] * xw[row], col, num_segments=N)
        out = out + dinv[:, None] * dinv[:, None] * xw
        return out + b

    def gnorm(h, w, b, a):
        ones = jnp.ones((N,), dtype=h.dtype)
        cnt = jnp.maximum(segment_sum(ones, batch, num_segments=G), 1.0)
        mean = segment_sum(h, batch, num_segments=G) / cnt[:, None]
        sub = h - a * mean[batch]
        var = segment_sum(sub * sub, batch, num_segments=G) / cnt[:, None]
        std = jnp.sqrt(var + EPS)
        return w * sub / std[batch] + b

    h = conv(x, W1, b1)
    h = gnorm(h, gn1_w, gn1_b, gn1_a)
    h = _prelu(h, p1)
    h = conv(h, W2, b2)
    h = gnorm(h, gn2_w, gn2_b, gn2_a)
    h = _prelu(h, p0)

    ones = jnp.ones((N,), dtype=h.dtype)
    cnt = jnp.maximum(segment_sum(ones, batch, num_segments=G), 1.0)
    x1 = segment_sum(h, batch, num_segments=G) / cnt[:, None]
    x2 = segment_max(h, batch, num_segments=G)
    x2 = jnp.where(jnp.isfinite(x2), x2, 0.0)
    pooled = jnp.concatenate([x1, x2], axis=1)

    out = _head(pooled, lin1_w, lin1_b, lin2_w, lin2_b, lin3_w, lin3_b, p0, p2)
    return jnp.squeeze(out)


# speedup vs baseline: 3.4827x; 2.7817x over previous
import functools

import jax
import jax.numpy as jnp
from jax import lax
from jax.experimental import pallas as pl
from jax.experimental.pallas import tpu as pltpu
from jax.experimental.pallas import tpu_sc as plsc

N = 50000
E = 800000
G = 128
EPS = 1e-5
NP = 50176
STRIPE = NP // 16
EPAD = 819200
EPT = EPAD // 16
EB = 512
SUB = EB // 128
CW = 32


def _sc_conv(nch, xv_flat, r, c, ew):
    mesh = plsc.VectorSubcoreMesh(core_axis_name="c", subcore_axis_name="s")
    npc = nch // 2

    def body(xv_hbm, r_hbm, c_hbm, ew_hbm, out_hbm,
             r_v, ew_v, gidx_v, dst_v, rows_v, acc_sh, sem):
        cid = lax.axis_index("c")
        sid = lax.axis_index("s")
        estart = sid * EPT

        for ch in range(npc):
            chunk = cid * npc + ch
            off = chunk * NP

            def zero_body(i, carry):
                for j in range(CW // 16):
                    rows_v[i, pl.ds(j * 16, 16)] = jnp.zeros((16,),
                                                             jnp.float32)
                return carry
            lax.fori_loop(0, EB, zero_body, 0, unroll=8)
            for z in range(STRIPE // EB):
                pltpu.sync_copy(
                    rows_v, acc_sh.at[pl.ds(sid * STRIPE + z * EB, EB)])
            rem = STRIPE % EB
            if rem:
                pltpu.sync_copy(
                    rows_v.at[pl.ds(0, rem)],
                    acc_sh.at[pl.ds(sid * STRIPE + (STRIPE // EB) * EB, rem)])
            plsc.subcore_barrier()

            def batch_body(b, carry):
                base = estart + b * EB
                pltpu.sync_copy(r_hbm.at[pl.ds(base, EB)], r_v)
                pltpu.sync_copy(ew_hbm.at[pl.ds(base, EB)], ew_v)
                for s in range(SUB):
                    pltpu.sync_copy(c_hbm.at[pl.ds(base + s * 128, 128)],
                                    dst_v.at[s])

                for k in range(EB // 16):
                    v = r_v[pl.ds(k * 16, 16)]
                    gidx_v[k // 8, pl.ds((k % 8) * 16, 16)] = v + off

                cps = [pltpu.async_copy(
                    xv_hbm.at[gidx_v.at[s]],
                    rows_v.at[pl.ds(s * 128, 128)], sem) for s in range(SUB)]
                for cp in cps:
                    cp.wait()

                dnums = lax.GatherDimensionNumbers(
                    offset_dims=(), collapsed_slice_dims=(0,),
                    start_index_map=(0,))

                def scale_body(k, carry2):
                    ewv = ew_v[pl.ds(k * 16, 16)]
                    for i in range(16):
                        rowb = k * 16 + i
                        spl = lax.gather(
                            ewv, jnp.full((16, 1), i, jnp.int32), dnums, (1,),
                            mode=lax.GatherScatterMode.PROMISE_IN_BOUNDS)
                        for j in range(2):
                            sl = (rowb, pl.ds(j * 16, 16))
                            rows_v[sl] = rows_v[sl] * spl
                    return carry2
                lax.fori_loop(0, EB // 16, scale_body, 0)

                for s in range(SUB):
                    pltpu.sync_copy(rows_v.at[pl.ds(s * 128, 128)],
                                    acc_sh.at[dst_v.at[s]], add=True)
                return carry
            lax.fori_loop(0, EPT // EB, batch_body, 0)

            plsc.subcore_barrier()
            pltpu.sync_copy(
                acc_sh.at[pl.ds(sid * STRIPE, STRIPE)],
                out_hbm.at[pl.ds(off + sid * STRIPE, STRIPE)])
            plsc.subcore_barrier()

    fn = pl.kernel(
        body,
        mesh=mesh,
        compiler_params=pltpu.CompilerParams(use_tc_tiling_on_sc=False),
        out_type=jax.ShapeDtypeStruct((nch * NP, CW), jnp.float32),
        scratch_types=[
            pltpu.VMEM((EB,), jnp.int32),
            pltpu.VMEM((EB,), jnp.float32),
            pltpu.VMEM((SUB, 128), jnp.int32),
            pltpu.VMEM((SUB, 128), jnp.int32),
            pltpu.VMEM((EB, CW), jnp.float32),
            pltpu.VMEM_SHARED((NP, CW), jnp.float32),
            pltpu.SemaphoreType.DMA,
        ],
    )
    return fn(xv_flat, r, c, ew)


def _proj_body(nch, x_ref, w_ref, dinv_ref, xv_ref):
    xw = jnp.dot(x_ref[...], w_ref[...],
                 preferred_element_type=jnp.float32,
                 precision=lax.Precision.HIGHEST)
    xv = xw * dinv_ref[...]
    for ch in range(nch):
        xv_ref[ch] = xv[:, ch * CW:(ch + 1) * CW]


def _tc_project(x, w, dinv):
    d_in = x.shape[1]
    nch = w.shape[1] // CW
    blk = 3136
    grid = NP // blk
    return pl.pallas_call(
        functools.partial(_proj_body, nch),
        grid=(grid,),
        in_specs=[
            pl.BlockSpec((blk, d_in), lambda i: (i, 0)),
            pl.BlockSpec((d_in, nch * CW), lambda i: (0, 0)),
            pl.BlockSpec((blk, 1), lambda i: (i, 0)),
        ],
        out_specs=pl.BlockSpec((nch, blk, CW), lambda i: (0, i, 0)),
        out_shape=jax.ShapeDtypeStruct((nch, NP, CW), jnp.float32),
    )(x, w, dinv)


def _head_body(pooled_ref, l1w_ref, l1b_ref, l2w_ref, l2b_ref, l3w_ref,
               l3b_ref, p0_ref, p2_ref, out_ref):
    h = pooled_ref[...]
    h = h @ l1w_ref[...] + l1b_ref[...][None, :]
    h = jnp.where(h >= 0, h, p0_ref[...][None, :] * h)
    h = h @ l2w_ref[...] + l2b_ref[...][None, :]
    h = jnp.where(h >= 0, h, p2_ref[...][None, :] * h)
    h = h @ l3w_ref[...] + l3b_ref[...][None, :]
    out_ref[...] = h


def _head(pooled, lin1_w, lin1_b, lin2_w, lin2_b, lin3_w, lin3_b, p0, p2):
    return pl.pallas_call(
        _head_body,
        out_shape=jax.ShapeDtypeStruct((G, 1), jnp.float32),
    )(pooled, lin1_w, lin1_b, lin2_w, lin2_b, lin3_w, lin3_b, p0, p2)


def kernel(x, edge_index, edge_attr, batch, W1, b1, W2, b2, gn1_w, gn1_b,
           gn1_a, gn2_w, gn2_b, gn2_a, p1, p0, p2, lin1_w, lin1_b, lin2_w,
           lin2_b, lin3_w, lin3_b):
    from jax.ops import segment_sum, segment_max

    row = edge_index[0]
    col = edge_index[1]

    x_pad = jnp.pad(x, ((0, NP - N), (0, 0)))
    batch_pad = jnp.pad(batch, (0, NP - N), constant_values=G)

    row_p = jnp.pad(row, (0, EPAD - E))
    col_p = jnp.pad(col, (0, EPAD - E))
    ew_p = jnp.pad(edge_attr, (0, EPAD - E))

    deg = segment_sum(edge_attr, col, num_segments=NP) + 1.0
    dinv = lax.rsqrt(deg)[:, None]

    def gnorm(h, w, b, a):
        ones = jnp.where(batch_pad < G, 1.0, 0.0)
        cnt = jnp.maximum(segment_sum(ones, batch_pad, num_segments=G), 1.0)
        mean = segment_sum(h * ones[:, None], batch_pad, num_segments=G) \
            / cnt[:, None]
        sub = h - a * mean[jnp.minimum(batch_pad, G - 1)]
        var = segment_sum(sub * sub * ones[:, None], batch_pad,
                          num_segments=G) / cnt[:, None]
        std = jnp.sqrt(var + EPS)
        return w * sub / std[jnp.minimum(batch_pad, G - 1)] + b

    def _prelu(h, w):
        return jnp.where(h >= 0, h, w * h)

    xv1 = _tc_project(x_pad, W1, dinv)
    acc1 = _sc_conv(8, xv1.reshape(8 * NP, CW), row_p, col_p, ew_p)
    h1 = dinv * (acc1.reshape(8, NP, CW) + xv1)
    h1 = jnp.moveaxis(h1, 0, 1).reshape(NP, 8 * CW) + b1

    h1 = _prelu(gnorm(h1, gn1_w, gn1_b, gn1_a), p1)

    xv2 = _tc_project(h1, W2, dinv)
    acc2 = _sc_conv(4, xv2.reshape(4 * NP, CW), row_p, col_p, ew_p)
    h2 = dinv * (acc2.reshape(4, NP, CW) + xv2)
    h2 = jnp.moveaxis(h2, 0, 1).reshape(NP, 4 * CW) + b2

    h2 = _prelu(gnorm(h2, gn2_w, gn2_b, gn2_a), p0)

    ones = jnp.where(batch_pad < G, 1.0, 0.0)
    cnt = jnp.maximum(segment_sum(ones, batch_pad, num_segments=G), 1.0)
    x1 = segment_sum(h2 * ones[:, None], batch_pad, num_segments=G) \
        / cnt[:, None]
    x2 = segment_max(h2, batch_pad, num_segments=G,
                     indices_are_sorted=True)
    x2 = jnp.where(jnp.isfinite(x2), x2, 0.0)
    pooled = jnp.concatenate([x1, x2], axis=1)

    out = _head(pooled, lin1_w, lin1_b, lin2_w, lin2_b, lin3_w, lin3_b,
                p0, p2)
    return jnp.squeeze(out)


# speedup vs baseline: 4.3143x; 1.2388x over previous
import functools

import jax
import jax.numpy as jnp
from jax import lax
from jax.experimental import pallas as pl
from jax.experimental.pallas import tpu as pltpu
from jax.experimental.pallas import tpu_sc as plsc

N = 50000
E = 800000
G = 128
EPS = 1e-5
NP = 50176
STRIPE = NP // 16
EPAD = 819200
EPT = EPAD // 16
EB = 512
SUB = EB // 128
CW = 32


def _sc_conv(nch, xv_flat, r, c, ew):
    mesh = plsc.VectorSubcoreMesh(core_axis_name="c", subcore_axis_name="s")
    npc = nch // 2

    def body(xv_hbm, r_hbm, c_hbm, ew_hbm, out_hbm,
             r_v, ew_v, gidx_v, dst_v, rows_v, acc_sh, sem):
        cid = lax.axis_index("c")
        sid = lax.axis_index("s")
        estart = sid * EPT

        for ch in range(npc):
            chunk = cid * npc + ch
            off = chunk * NP

            def zero_body(i, carry):
                for j in range(CW // 16):
                    rows_v[i, pl.ds(j * 16, 16)] = jnp.zeros((16,),
                                                             jnp.float32)
                return carry
            lax.fori_loop(0, EB, zero_body, 0, unroll=8)
            for z in range(STRIPE // EB):
                pltpu.sync_copy(
                    rows_v, acc_sh.at[pl.ds(sid * STRIPE + z * EB, EB)])
            rem = STRIPE % EB
            if rem:
                pltpu.sync_copy(
                    rows_v.at[pl.ds(0, rem)],
                    acc_sh.at[pl.ds(sid * STRIPE + (STRIPE // EB) * EB, rem)])
            plsc.subcore_barrier()

            def batch_body(b, carry):
                base = estart + b * EB
                pltpu.sync_copy(r_hbm.at[pl.ds(base, EB)], r_v)
                pltpu.sync_copy(ew_hbm.at[pl.ds(base, EB)], ew_v)
                for s in range(SUB):
                    pltpu.sync_copy(c_hbm.at[pl.ds(base + s * 128, 128)],
                                    dst_v.at[s])

                for k in range(EB // 16):
                    v = r_v[pl.ds(k * 16, 16)]
                    gidx_v[k // 8, pl.ds((k % 8) * 16, 16)] = v + off

                cps = [pltpu.async_copy(
                    xv_hbm.at[gidx_v.at[s]],
                    rows_v.at[pl.ds(s * 128, 128)], sem) for s in range(SUB)]
                for cp in cps:
                    cp.wait()

                dnums = lax.GatherDimensionNumbers(
                    offset_dims=(), collapsed_slice_dims=(0,),
                    start_index_map=(0,))

                def scale_body(k, carry2):
                    ewv = ew_v[pl.ds(k * 16, 16)]
                    for i in range(16):
                        rowb = k * 16 + i
                        spl = lax.gather(
                            ewv, jnp.full((16, 1), i, jnp.int32), dnums, (1,),
                            mode=lax.GatherScatterMode.PROMISE_IN_BOUNDS)
                        for j in range(2):
                            sl = (rowb, pl.ds(j * 16, 16))
                            rows_v[sl] = rows_v[sl] * spl
                    return carry2
                lax.fori_loop(0, EB // 16, scale_body, 0)

                for s in range(SUB):
                    pltpu.sync_copy(rows_v.at[pl.ds(s * 128, 128)],
                                    acc_sh.at[dst_v.at[s]], add=True)
                return carry
            lax.fori_loop(0, EPT // EB, batch_body, 0)

            plsc.subcore_barrier()
            pltpu.sync_copy(
                acc_sh.at[pl.ds(sid * STRIPE, STRIPE)],
                out_hbm.at[pl.ds(off + sid * STRIPE, STRIPE)])
            plsc.subcore_barrier()

    fn = pl.kernel(
        body,
        mesh=mesh,
        compiler_params=pltpu.CompilerParams(use_tc_tiling_on_sc=False),
        out_type=jax.ShapeDtypeStruct((nch * NP, CW), jnp.float32),
        scratch_types=[
            pltpu.VMEM((EB,), jnp.int32),
            pltpu.VMEM((EB,), jnp.float32),
            pltpu.VMEM((SUB, 128), jnp.int32),
            pltpu.VMEM((SUB, 128), jnp.int32),
            pltpu.VMEM((EB, CW), jnp.float32),
            pltpu.VMEM_SHARED((NP, CW), jnp.float32),
            pltpu.SemaphoreType.DMA,
        ],
    )
    return fn(xv_flat, r, c, ew)


def _proj_body(nch, x_ref, w_ref, dinv_ref, xv_ref):
    xw = jnp.dot(x_ref[...], w_ref[...],
                 preferred_element_type=jnp.float32,
                 precision=lax.Precision.HIGHEST)
    xv = xw * dinv_ref[...]
    for ch in range(nch):
        xv_ref[ch] = xv[:, ch * CW:(ch + 1) * CW]


def _tc_project(x, w, dinv):
    d_in = x.shape[1]
    nch = w.shape[1] // CW
    blk = 3136
    grid = NP // blk
    return pl.pallas_call(
        functools.partial(_proj_body, nch),
        grid=(grid,),
        in_specs=[
            pl.BlockSpec((blk, d_in), lambda i: (i, 0)),
            pl.BlockSpec((d_in, nch * CW), lambda i: (0, 0)),
            pl.BlockSpec((blk, 1), lambda i: (i, 0)),
        ],
        out_specs=pl.BlockSpec((nch, blk, CW), lambda i: (0, i, 0)),
        out_shape=jax.ShapeDtypeStruct((nch, NP, CW), jnp.float32),
    )(x, w, dinv)


BLK = 1568


def _stats_body(nch, acc_ref, xv_ref, dinv_ref, b_ref, batch_ref,
                h_ref, s1_ref, s2_ref, cnt_ref):
    f = nch * CW
    hb = jnp.concatenate(
        [acc_ref[ch] + xv_ref[ch] for ch in range(nch)], axis=1)
    hb = dinv_ref[...] * hb + b_ref[...][None, :]
    h_ref[...] = hb

    gids = jax.lax.broadcasted_iota(jnp.int32, (BLK, G), 1)
    p = (batch_ref[...] == gids).astype(jnp.float32)

    @pl.when(pl.program_id(0) == 0)
    def _():
        s1_ref[...] = jnp.zeros_like(s1_ref)
        s2_ref[...] = jnp.zeros_like(s2_ref)
        cnt_ref[...] = jnp.zeros_like(cnt_ref)

    dn = (((0,), (0,)), ((), ()))
    s1_ref[...] += lax.dot_general(p, hb, dn,
                                   preferred_element_type=jnp.float32,
                                   precision=lax.Precision.HIGHEST)
    s2_ref[...] += lax.dot_general(p, hb * hb, dn,
                                   preferred_element_type=jnp.float32,
                                   precision=lax.Precision.HIGHEST)
    cnt_ref[...] += jnp.sum(p, axis=0, keepdims=True)


def _tc_stats(nch, acc, xv, dinv, b, batch2d):
    f = nch * CW
    return pl.pallas_call(
        functools.partial(_stats_body, nch),
        grid=(NP // BLK,),
        in_specs=[
            pl.BlockSpec((nch, BLK, CW), lambda i: (0, i, 0)),
            pl.BlockSpec((nch, BLK, CW), lambda i: (0, i, 0)),
            pl.BlockSpec((BLK, 1), lambda i: (i, 0)),
            pl.BlockSpec((f,), lambda i: (0,)),
            pl.BlockSpec((BLK, 1), lambda i: (i, 0)),
        ],
        out_specs=[
            pl.BlockSpec((BLK, f), lambda i: (i, 0)),
            pl.BlockSpec((G, f), lambda i: (0, 0)),
            pl.BlockSpec((G, f), lambda i: (0, 0)),
            pl.BlockSpec((1, G), lambda i: (0, 0)),
        ],
        out_shape=[
            jax.ShapeDtypeStruct((NP, f), jnp.float32),
            jax.ShapeDtypeStruct((G, f), jnp.float32),
            jax.ShapeDtypeStruct((G, f), jnp.float32),
            jax.ShapeDtypeStruct((1, G), jnp.float32),
        ],
    )(acc, xv, dinv, b, batch2d)


def _apply_body(nch_out, *refs):
    (h_ref, batch_ref, s1_ref, s2_ref, cnt_ref, w_ref, b_ref, a_ref,
     prelu_ref, dinv_ref) = refs[:10]
    if nch_out:
        w2_ref, out_ref = refs[10:]
    else:
        (out_ref,) = refs[10:]
    cntc = jnp.maximum(cnt_ref[...], 1.0).T
    mean = s1_ref[...] / cntc
    m2 = s2_ref[...] / cntc
    a = a_ref[...][None, :]
    w = w_ref[...][None, :]
    var = m2 - (2.0 * a - a * a) * mean * mean
    std = jnp.sqrt(var + EPS)
    scale = w / std
    shift = b_ref[...][None, :] - a * scale * mean

    gids = jax.lax.broadcasted_iota(jnp.int32, (BLK, G), 1)
    p = (batch_ref[...] == gids).astype(jnp.float32)
    scale_n = jnp.dot(p, scale, preferred_element_type=jnp.float32,
                      precision=lax.Precision.HIGHEST)
    shift_n = jnp.dot(p, shift, preferred_element_type=jnp.float32,
                      precision=lax.Precision.HIGHEST)
    act = scale_n * h_ref[...] + shift_n
    act = jnp.where(act >= 0, act, prelu_ref[...][None, :] * act)

    if nch_out:
        yv = jnp.dot(act, w2_ref[...], preferred_element_type=jnp.float32,
                     precision=lax.Precision.HIGHEST) * dinv_ref[...]
        for ch in range(nch_out):
            out_ref[ch] = yv[:, ch * CW:(ch + 1) * CW]
    else:
        out_ref[...] = act


def _tc_apply(nch_out, h, batch2d, s1, s2, cnt, w, b, a, prelu, dinv, w2):
    f = h.shape[1]
    f2 = w2.shape[1] if nch_out else f
    if nch_out:
        out_spec = pl.BlockSpec((nch_out, BLK, CW), lambda i: (0, i, 0))
        out_shape = jax.ShapeDtypeStruct((nch_out, NP, CW), jnp.float32)
    else:
        out_spec = pl.BlockSpec((BLK, f), lambda i: (i, 0))
        out_shape = jax.ShapeDtypeStruct((NP, f), jnp.float32)
    in_specs = [
        pl.BlockSpec((BLK, f), lambda i: (i, 0)),
        pl.BlockSpec((BLK, 1), lambda i: (i, 0)),
        pl.BlockSpec((G, f), lambda i: (0, 0)),
        pl.BlockSpec((G, f), lambda i: (0, 0)),
        pl.BlockSpec((1, G), lambda i: (0, 0)),
        pl.BlockSpec((f,), lambda i: (0,)),
        pl.BlockSpec((f,), lambda i: (0,)),
        pl.BlockSpec((f,), lambda i: (0,)),
        pl.BlockSpec((f,), lambda i: (0,)),
        pl.BlockSpec((BLK, 1), lambda i: (i, 0)),
    ]
    args = [h, batch2d, s1, s2, cnt, w, b, a, prelu, dinv]
    if nch_out:
        in_specs.append(pl.BlockSpec((f, f2), lambda i: (0, 0)))
        args.append(w2)
    return pl.pallas_call(
        functools.partial(_apply_body, nch_out),
        grid=(NP // BLK,),
        in_specs=in_specs,
        out_specs=out_spec,
        out_shape=out_shape,
    )(*args)


def _head_body(pooled_ref, l1w_ref, l1b_ref, l2w_ref, l2b_ref, l3w_ref,
               l3b_ref, p0_ref, p2_ref, out_ref):
    h = pooled_ref[...]
    h = h @ l1w_ref[...] + l1b_ref[...][None, :]
    h = jnp.where(h >= 0, h, p0_ref[...][None, :] * h)
    h = h @ l2w_ref[...] + l2b_ref[...][None, :]
    h = jnp.where(h >= 0, h, p2_ref[...][None, :] * h)
    h = h @ l3w_ref[...] + l3b_ref[...][None, :]
    out_ref[...] = h


def _head(pooled, lin1_w, lin1_b, lin2_w, lin2_b, lin3_w, lin3_b, p0, p2):
    return pl.pallas_call(
        _head_body,
        out_shape=jax.ShapeDtypeStruct((G, 1), jnp.float32),
    )(pooled, lin1_w, lin1_b, lin2_w, lin2_b, lin3_w, lin3_b, p0, p2)


def kernel(x, edge_index, edge_attr, batch, W1, b1, W2, b2, gn1_w, gn1_b,
           gn1_a, gn2_w, gn2_b, gn2_a, p1, p0, p2, lin1_w, lin1_b, lin2_w,
           lin2_b, lin3_w, lin3_b):
    from jax.ops import segment_sum, segment_max

    row = edge_index[0]
    col = edge_index[1]

    x_pad = jnp.pad(x, ((0, NP - N), (0, 0)))
    batch_pad = jnp.pad(batch, (0, NP - N), constant_values=G)

    row_p = jnp.pad(row, (0, EPAD - E))
    col_p = jnp.pad(col, (0, EPAD - E))
    ew_p = jnp.pad(edge_attr, (0, EPAD - E))

    deg = segment_sum(edge_attr, col, num_segments=NP) + 1.0
    dinv = lax.rsqrt(deg)[:, None]
    batch2d = batch_pad[:, None]

    xv1 = _tc_project(x_pad, W1, dinv)
    acc1 = _sc_conv(8, xv1.reshape(8 * NP, CW), row_p, col_p, ew_p)
    h1r, s1a, s2a, cnt1 = _tc_stats(8, acc1.reshape(8, NP, CW), xv1,
                                    dinv, b1, batch2d)
    yv2 = _tc_apply(4, h1r, batch2d, s1a, s2a, cnt1, gn1_w, gn1_b, gn1_a,
                    p1, dinv, W2)

    acc2 = _sc_conv(4, yv2.reshape(4 * NP, CW), row_p, col_p, ew_p)
    h2r, s1b, s2b, cnt2 = _tc_stats(4, acc2.reshape(4, NP, CW), yv2,
                                    dinv, b2, batch2d)
    h2f = _tc_apply(0, h2r, batch2d, s1b, s2b, cnt2, gn2_w, gn2_b, gn2_a,
                    p0, dinv, None)

    ones = jnp.where(batch_pad < G, 1.0, 0.0)
    cnt = jnp.maximum(cnt1[0], 1.0)
    x1 = segment_sum(h2f * ones[:, None], batch_pad, num_segments=G) \
        / cnt[:, None]
    x2 = segment_max(h2f, batch_pad, num_segments=G,
                     indices_are_sorted=True)
    x2 = jnp.where(jnp.isfinite(x2), x2, 0.0)
    pooled = jnp.concatenate([x1, x2], axis=1)

    out = _head(pooled, lin1_w, lin1_b, lin2_w, lin2_b, lin3_w, lin3_b,
                p0, p2)
    return jnp.squeeze(out)


# speedup vs baseline: 4.8913x; 1.1337x over previous
import functools

import jax
import jax.numpy as jnp
from jax import lax
from jax.experimental import pallas as pl
from jax.experimental.pallas import tpu as pltpu
from jax.experimental.pallas import tpu_sc as plsc

N = 50000
E = 800000
G = 128
EPS = 1e-5
NP = 50176
STRIPE = NP // 16
EPAD = 819200
EPT = EPAD // 16
EB = 512
SUB = EB // 128
CW = 32


def _sc_conv(nch, xv_flat, r, c, ew):
    mesh = plsc.VectorSubcoreMesh(core_axis_name="c", subcore_axis_name="s")
    npc = nch // 2

    def body(xv_hbm, r_hbm, c_hbm, ew_hbm, out_hbm,
             r_v, ew_v, gidx_v, dst_v, rows_v, acc_sh, sem):
        cid = lax.axis_index("c")
        sid = lax.axis_index("s")
        estart = sid * EPT

        for ch in range(npc):
            chunk = cid * npc + ch
            off = chunk * NP

            def zero_body(i, carry):
                for j in range(CW // 16):
                    rows_v[i, pl.ds(j * 16, 16)] = jnp.zeros((16,),
                                                             jnp.float32)
                return carry
            lax.fori_loop(0, EB, zero_body, 0, unroll=8)
            for z in range(STRIPE // EB):
                pltpu.sync_copy(
                    rows_v, acc_sh.at[pl.ds(sid * STRIPE + z * EB, EB)])
            rem = STRIPE % EB
            if rem:
                pltpu.sync_copy(
                    rows_v.at[pl.ds(0, rem)],
                    acc_sh.at[pl.ds(sid * STRIPE + (STRIPE // EB) * EB, rem)])
            plsc.subcore_barrier()

            def batch_body(b, carry):
                base = estart + b * EB
                pltpu.sync_copy(r_hbm.at[pl.ds(base, EB)], r_v)
                pltpu.sync_copy(ew_hbm.at[pl.ds(base, EB)], ew_v)
                for s in range(SUB):
                    pltpu.sync_copy(c_hbm.at[pl.ds(base + s * 128, 128)],
                                    dst_v.at[s])

                for k in range(EB // 16):
                    v = r_v[pl.ds(k * 16, 16)]
                    gidx_v[k // 8, pl.ds((k % 8) * 16, 16)] = v + off

                cps = [pltpu.async_copy(
                    xv_hbm.at[gidx_v.at[s]],
                    rows_v.at[pl.ds(s * 128, 128)], sem) for s in range(SUB)]
                for cp in cps:
                    cp.wait()

                dnums = lax.GatherDimensionNumbers(
                    offset_dims=(), collapsed_slice_dims=(0,),
                    start_index_map=(0,))

                def scale_body(k, carry2):
                    ewv = ew_v[pl.ds(k * 16, 16)]
                    for i in range(16):
                        rowb = k * 16 + i
                        spl = lax.gather(
                            ewv, jnp.full((16, 1), i, jnp.int32), dnums, (1,),
                            mode=lax.GatherScatterMode.PROMISE_IN_BOUNDS)
                        for j in range(2):
                            sl = (rowb, pl.ds(j * 16, 16))
                            rows_v[sl] = rows_v[sl] * spl
                    return carry2
                lax.fori_loop(0, EB // 16, scale_body, 0)

                for s in range(SUB):
                    pltpu.sync_copy(rows_v.at[pl.ds(s * 128, 128)],
                                    acc_sh.at[dst_v.at[s]], add=True)
                return carry
            lax.fori_loop(0, EPT // EB, batch_body, 0)

            plsc.subcore_barrier()
            pltpu.sync_copy(
                acc_sh.at[pl.ds(sid * STRIPE, STRIPE)],
                out_hbm.at[pl.ds(off + sid * STRIPE, STRIPE)])
            plsc.subcore_barrier()

    fn = pl.kernel(
        body,
        mesh=mesh,
        compiler_params=pltpu.CompilerParams(use_tc_tiling_on_sc=False),
        out_type=jax.ShapeDtypeStruct((nch * NP, CW), jnp.float32),
        scratch_types=[
            pltpu.VMEM((EB,), jnp.int32),
            pltpu.VMEM((EB,), jnp.float32),
            pltpu.VMEM((SUB, 128), jnp.int32),
            pltpu.VMEM((SUB, 128), jnp.int32),
            pltpu.VMEM((EB, CW), jnp.float32),
            pltpu.VMEM_SHARED((NP, CW), jnp.float32),
            pltpu.SemaphoreType.DMA,
        ],
    )
    return fn(xv_flat, r, c, ew)


def _sc_deg(c, ew):
    mesh = plsc.VectorSubcoreMesh(core_axis_name="c", subcore_axis_name="s")
    ept = EPAD // 32

    def body(c_hbm, ew_hbm, out_hbm, ew_v, dst_v, stage_v, acc_sh, sem):
        cid = lax.axis_index("c")
        sid = lax.axis_index("s")
        wid = sid * 2 + cid
        estart = wid * ept

        lane0 = lax.iota(jnp.int32, 16) == 0

        def zs(i, carry):
            stage_v[i, pl.ds(0, 16)] = jnp.zeros((16,), jnp.float32)
            return carry
        lax.fori_loop(0, EB, zs, 0, unroll=8)
        for z in range(STRIPE // EB):
            pltpu.sync_copy(stage_v,
                            acc_sh.at[pl.ds(sid * STRIPE + z * EB, EB)])
        rem = STRIPE % EB
        if rem:
            pltpu.sync_copy(
                stage_v.at[pl.ds(0, rem)],
                acc_sh.at[pl.ds(sid * STRIPE + (STRIPE // EB) * EB, rem)])
        plsc.subcore_barrier()

        dnums = lax.GatherDimensionNumbers(
            offset_dims=(), collapsed_slice_dims=(0,), start_index_map=(0,))

        def batch_body(b, carry):
            base = estart + b * EB
            pltpu.sync_copy(ew_hbm.at[pl.ds(base, EB)], ew_v)
            for s in range(SUB):
                pltpu.sync_copy(c_hbm.at[pl.ds(base + s * 128, 128)],
                                dst_v.at[s])

            def wcol(k, carry2):
                ewv = ew_v[pl.ds(k * 16, 16)]
                for i in range(16):
                    spl = lax.gather(
                        ewv, jnp.full((16, 1), i, jnp.int32), dnums, (1,),
                        mode=lax.GatherScatterMode.PROMISE_IN_BOUNDS)
                    stage_v[k * 16 + i, pl.ds(0, 16)] = jnp.where(
                        lane0, spl, 0.0)
                return carry2
            lax.fori_loop(0, EB // 16, wcol, 0)

            for s in range(SUB):
                pltpu.sync_copy(stage_v.at[pl.ds(s * 128, 128)],
                                acc_sh.at[dst_v.at[s]], add=True)
            return carry
        lax.fori_loop(0, ept // EB, batch_body, 0)

        plsc.subcore_barrier()
        pltpu.sync_copy(
            acc_sh.at[pl.ds(sid * STRIPE, STRIPE)],
            out_hbm.at[pl.ds(cid * NP + sid * STRIPE, STRIPE)])

    fn = pl.kernel(
        body,
        mesh=mesh,
        compiler_params=pltpu.CompilerParams(use_tc_tiling_on_sc=False),
        out_type=jax.ShapeDtypeStruct((2 * NP, 16), jnp.float32),
        scratch_types=[
            pltpu.VMEM((EB,), jnp.float32),
            pltpu.VMEM((SUB, 128), jnp.int32),
            pltpu.VMEM((EB, 16), jnp.float32),
            pltpu.VMEM_SHARED((NP, 16), jnp.float32),
            pltpu.SemaphoreType.DMA,
        ],
    )
    return fn(c, ew)


NEG = -3.0e38


def _pool_body(h_ref, batch_ref, sum_ref, max_ref):
    @pl.when(pl.program_id(0) == 0)
    def _():
        sum_ref[...] = jnp.zeros_like(sum_ref)
        max_ref[...] = jnp.full_like(max_ref, NEG)

    hb = h_ref[...]
    bb = batch_ref[...]
    gids = jax.lax.broadcasted_iota(jnp.int32, (BLK, G + 1), 1)
    p = (bb == gids).astype(jnp.float32)
    sum_ref[...] += lax.dot_general(
        p, hb, (((0,), (0,)), ((), ())),
        preferred_element_type=jnp.float32,
        precision=lax.Precision.HIGHEST)

    gmin = jnp.min(bb)
    gmax = jnp.max(bb)

    def gbody(g, carry):
        masked = jnp.where(bb == g, hb, NEG)
        mx = jnp.max(masked, axis=0, keepdims=True)
        cur = max_ref[pl.ds(g, 1), :]
        max_ref[pl.ds(g, 1), :] = jnp.maximum(cur, mx)
        return carry
    lax.fori_loop(gmin, gmax + 1, gbody, 0)


def _tc_pool(h2f, batch2d):
    return pl.pallas_call(
        _pool_body,
        grid=(NP // BLK,),
        in_specs=[
            pl.BlockSpec((BLK, 128), lambda i: (i, 0)),
            pl.BlockSpec((BLK, 1), lambda i: (i, 0)),
        ],
        out_specs=[
            pl.BlockSpec((G + 1, 128), lambda i: (0, 0)),
            pl.BlockSpec((G + 1, 128), lambda i: (0, 0)),
        ],
        out_shape=[
            jax.ShapeDtypeStruct((G + 1, 128), jnp.float32),
            jax.ShapeDtypeStruct((G + 1, 128), jnp.float32),
        ],
    )(h2f, batch2d)


def _proj_body(nch, x_ref, w_ref, degp_ref, xv_ref, dinv_ref):
    deg = degp_ref[0, :, 0:1] + degp_ref[1, :, 0:1] + 1.0
    dinv = lax.rsqrt(deg)
    dinv_ref[...] = dinv
    xw = jnp.dot(x_ref[...], w_ref[...],
                 preferred_element_type=jnp.float32,
                 precision=lax.Precision.HIGHEST)
    xv = xw * dinv
    for ch in range(nch):
        xv_ref[ch] = xv[:, ch * CW:(ch + 1) * CW]


def _tc_project(x, w, degp):
    d_in = x.shape[1]
    nch = w.shape[1] // CW
    blk = 3136
    grid = NP // blk
    return pl.pallas_call(
        functools.partial(_proj_body, nch),
        grid=(grid,),
        in_specs=[
            pl.BlockSpec((blk, d_in), lambda i: (i, 0)),
            pl.BlockSpec((d_in, nch * CW), lambda i: (0, 0)),
            pl.BlockSpec((2, blk, 16), lambda i: (0, i, 0)),
        ],
        out_specs=[
            pl.BlockSpec((nch, blk, CW), lambda i: (0, i, 0)),
            pl.BlockSpec((blk, 1), lambda i: (i, 0)),
        ],
        out_shape=[
            jax.ShapeDtypeStruct((nch, NP, CW), jnp.float32),
            jax.ShapeDtypeStruct((NP, 1), jnp.float32),
        ],
    )(x, w, degp)


BLK = 1568


def _stats_body(nch, acc_ref, xv_ref, dinv_ref, b_ref, batch_ref,
                h_ref, s1_ref, s2_ref, cnt_ref):
    f = nch * CW
    hb = jnp.concatenate(
        [acc_ref[ch] + xv_ref[ch] for ch in range(nch)], axis=1)
    hb = dinv_ref[...] * hb + b_ref[...][None, :]
    h_ref[...] = hb

    gids = jax.lax.broadcasted_iota(jnp.int32, (BLK, G), 1)
    p = (batch_ref[...] == gids).astype(jnp.float32)

    @pl.when(pl.program_id(0) == 0)
    def _():
        s1_ref[...] = jnp.zeros_like(s1_ref)
        s2_ref[...] = jnp.zeros_like(s2_ref)
        cnt_ref[...] = jnp.zeros_like(cnt_ref)

    dn = (((0,), (0,)), ((), ()))
    s1_ref[...] += lax.dot_general(p, hb, dn,
                                   preferred_element_type=jnp.float32,
                                   precision=lax.Precision.HIGHEST)
    s2_ref[...] += lax.dot_general(p, hb * hb, dn,
                                   preferred_element_type=jnp.float32,
                                   precision=lax.Precision.HIGHEST)
    cnt_ref[...] += jnp.sum(p, axis=0, keepdims=True)


def _tc_stats(nch, acc, xv, dinv, b, batch2d):
    f = nch * CW
    return pl.pallas_call(
        functools.partial(_stats_body, nch),
        grid=(NP // BLK,),
        in_specs=[
            pl.BlockSpec((nch, BLK, CW), lambda i: (0, i, 0)),
            pl.BlockSpec((nch, BLK, CW), lambda i: (0, i, 0)),
            pl.BlockSpec((BLK, 1), lambda i: (i, 0)),
            pl.BlockSpec((f,), lambda i: (0,)),
            pl.BlockSpec((BLK, 1), lambda i: (i, 0)),
        ],
        out_specs=[
            pl.BlockSpec((BLK, f), lambda i: (i, 0)),
            pl.BlockSpec((G, f), lambda i: (0, 0)),
            pl.BlockSpec((G, f), lambda i: (0, 0)),
            pl.BlockSpec((1, G), lambda i: (0, 0)),
        ],
        out_shape=[
            jax.ShapeDtypeStruct((NP, f), jnp.float32),
            jax.ShapeDtypeStruct((G, f), jnp.float32),
            jax.ShapeDtypeStruct((G, f), jnp.float32),
            jax.ShapeDtypeStruct((1, G), jnp.float32),
        ],
    )(acc, xv, dinv, b, batch2d)


def _apply_body(nch_out, *refs):
    (h_ref, batch_ref, s1_ref, s2_ref, cnt_ref, w_ref, b_ref, a_ref,
     prelu_ref, dinv_ref) = refs[:10]
    if nch_out:
        w2_ref, out_ref = refs[10:]
    else:
        (out_ref,) = refs[10:]
    cntc = jnp.maximum(cnt_ref[...], 1.0).T
    mean = s1_ref[...] / cntc
    m2 = s2_ref[...] / cntc
    a = a_ref[...][None, :]
    w = w_ref[...][None, :]
    var = m2 - (2.0 * a - a * a) * mean * mean
    std = jnp.sqrt(var + EPS)
    scale = w / std
    shift = b_ref[...][None, :] - a * scale * mean

    gids = jax.lax.broadcasted_iota(jnp.int32, (BLK, G), 1)
    p = (batch_ref[...] == gids).astype(jnp.float32)
    scale_n = jnp.dot(p, scale, preferred_element_type=jnp.float32,
                      precision=lax.Precision.HIGHEST)
    shift_n = jnp.dot(p, shift, preferred_element_type=jnp.float32,
                      precision=lax.Precision.HIGHEST)
    act = scale_n * h_ref[...] + shift_n
    act = jnp.where(act >= 0, act, prelu_ref[...][None, :] * act)

    if nch_out:
        yv = jnp.dot(act, w2_ref[...], preferred_element_type=jnp.float32,
                     precision=lax.Precision.HIGHEST) * dinv_ref[...]
        for ch in range(nch_out):
            out_ref[ch] = yv[:, ch * CW:(ch + 1) * CW]
    else:
        out_ref[...] = act


def _tc_apply(nch_out, h, batch2d, s1, s2, cnt, w, b, a, prelu, dinv, w2):
    f = h.shape[1]
    f2 = w2.shape[1] if nch_out else f
    if nch_out:
        out_spec = pl.BlockSpec((nch_out, BLK, CW), lambda i: (0, i, 0))
        out_shape = jax.ShapeDtypeStruct((nch_out, NP, CW), jnp.float32)
    else:
        out_spec = pl.BlockSpec((BLK, f), lambda i: (i, 0))
        out_shape = jax.ShapeDtypeStruct((NP, f), jnp.float32)
    in_specs = [
        pl.BlockSpec((BLK, f), lambda i: (i, 0)),
        pl.BlockSpec((BLK, 1), lambda i: (i, 0)),
        pl.BlockSpec((G, f), lambda i: (0, 0)),
        pl.BlockSpec((G, f), lambda i: (0, 0)),
        pl.BlockSpec((1, G), lambda i: (0, 0)),
        pl.BlockSpec((f,), lambda i: (0,)),
        pl.BlockSpec((f,), lambda i: (0,)),
        pl.BlockSpec((f,), lambda i: (0,)),
        pl.BlockSpec((f,), lambda i: (0,)),
        pl.BlockSpec((BLK, 1), lambda i: (i, 0)),
    ]
    args = [h, batch2d, s1, s2, cnt, w, b, a, prelu, dinv]
    if nch_out:
        in_specs.append(pl.BlockSpec((f, f2), lambda i: (0, 0)))
        args.append(w2)
    return pl.pallas_call(
        functools.partial(_apply_body, nch_out),
        grid=(NP // BLK,),
        in_specs=in_specs,
        out_specs=out_spec,
        out_shape=out_shape,
    )(*args)


def _head_body(sum_ref, max_ref, cnt_ref, l1w_ref, l1b_ref, l2w_ref,
               l2b_ref, l3w_ref, l3b_ref, p0_ref, p2_ref, out_ref):
    psum = sum_ref[0:G]
    pmax = max_ref[0:G]
    cntc = jnp.maximum(cnt_ref[...], 1.0).T
    mean = psum / cntc
    pmax = jnp.where(cnt_ref[...].T > 0, pmax, 0.0)
    h = jnp.concatenate([mean, pmax], axis=1)
    h = h @ l1w_ref[...] + l1b_ref[...][None, :]
    h = jnp.where(h >= 0, h, p0_ref[...][None, :] * h)
    h = h @ l2w_ref[...] + l2b_ref[...][None, :]
    h = jnp.where(h >= 0, h, p2_ref[...][None, :] * h)
    h = h @ l3w_ref[...] + l3b_ref[...][None, :]
    out_ref[...] = h


def _head(psum, pmax, cnt, lin1_w, lin1_b, lin2_w, lin2_b, lin3_w,
          lin3_b, p0, p2):
    return pl.pallas_call(
        _head_body,
        out_shape=jax.ShapeDtypeStruct((G, 1), jnp.float32),
    )(psum, pmax, cnt, lin1_w, lin1_b, lin2_w, lin2_b, lin3_w, lin3_b,
      p0, p2)


def kernel(x, edge_index, edge_attr, batch, W1, b1, W2, b2, gn1_w, gn1_b,
           gn1_a, gn2_w, gn2_b, gn2_a, p1, p0, p2, lin1_w, lin1_b, lin2_w,
           lin2_b, lin3_w, lin3_b):
    from jax.ops import segment_sum, segment_max

    row = edge_index[0]
    col = edge_index[1]

    x_pad = jnp.pad(x, ((0, NP - N), (0, 0)))
    batch_pad = jnp.pad(batch, (0, NP - N), constant_values=G)

    row_p = jnp.pad(row, (0, EPAD - E))
    col_p = jnp.pad(col, (0, EPAD - E))
    ew_p = jnp.pad(edge_attr, (0, EPAD - E))

    batch2d = batch_pad[:, None]

    degp = _sc_deg(col_p, ew_p)

    xv1, dinv = _tc_project(x_pad, W1, degp.reshape(2, NP, 16))
    acc1 = _sc_conv(8, xv1.reshape(8 * NP, CW), row_p, col_p, ew_p)
    h1r, s1a, s2a, cnt1 = _tc_stats(8, acc1.reshape(8, NP, CW), xv1,
                                    dinv, b1, batch2d)
    yv2 = _tc_apply(4, h1r, batch2d, s1a, s2a, cnt1, gn1_w, gn1_b, gn1_a,
                    p1, dinv, W2)

    acc2 = _sc_conv(4, yv2.reshape(4 * NP, CW), row_p, col_p, ew_p)
    h2r, s1b, s2b, cnt2 = _tc_stats(4, acc2.reshape(4, NP, CW), yv2,
                                    dinv, b2, batch2d)
    h2f = _tc_apply(0, h2r, batch2d, s1b, s2b, cnt2, gn2_w, gn2_b, gn2_a,
                    p0, dinv, None)

    psum, pmax = _tc_pool(h2f, batch2d)
    out = _head(psum, pmax, cnt1, lin1_w, lin1_b, lin2_w, lin2_b, lin3_w,
                lin3_b, p0, p2)
    return jnp.squeeze(out)
